# Initial kernel scaffold; baseline (speedup 1.0000x reference)
import jax
import jax.numpy as jnp
from jax.experimental import pallas as pl


def kernel(x, edge_index, edge_type, W, W_root, bias):
    raise NotImplementedError("write your pallas kernel here")


# speedup vs baseline: 17.2422x; 17.2422x over previous
import functools

import jax
import jax.numpy as jnp
import numpy as np
from jax import lax
from jax.experimental import pallas as pl
from jax.experimental.pallas import tpu as pltpu
from jax.experimental.pallas import tpu_sc as plsc

N = 10000
E = 320000
D = 128
R = 8

NC = 2
NS = 16
NW = NC * NS
L = 16

EP = E // NW
SEG = N * R
MC = 80
ARS = N // NS


_mesh = functools.partial(
    plsc.VectorSubcoreMesh,
    core_axis_name="c",
    subcore_axis_name="s",
    num_cores=NC,
    num_subcores=NS,
)


@functools.partial(
    pl.kernel,
    out_type=jax.ShapeDtypeStruct((NW * SEG,), jnp.float32),
    mesh=_mesh(),
    compiler_params=pltpu.CompilerParams(needs_layout_passes=False),
    scratch_types=[
        pltpu.VMEM((EP,), jnp.int32),
        pltpu.VMEM((EP,), jnp.int32),
        pltpu.VMEM((SEG,), jnp.float32),
    ],
)
def _count_kernel(dst, et, cnt_out, dst_v, et_v, cnt_v):
    c = lax.axis_index("c")
    s = lax.axis_index("s")
    wid = s * NC + c
    base = wid * EP

    def zero(i, _):
        cnt_v[pl.ds(i * L, L)] = jnp.zeros((L,), jnp.float32)
        return 0

    lax.fori_loop(0, SEG // L, zero, 0)
    pltpu.sync_copy(dst.at[pl.ds(base, EP)], dst_v)
    pltpu.sync_copy(et.at[pl.ds(base, EP)], et_v)

    ones16 = jnp.ones((L,), jnp.float32)

    def lane(i, _):
        off = i * L
        d = dst_v[pl.ds(off, L)]
        t = et_v[pl.ds(off, L)]
        plsc.addupdate_scatter(cnt_v, [d * R + t], ones16)
        return 0

    lax.fori_loop(0, EP // L, lane, 0)
    pltpu.sync_copy(cnt_v, cnt_out.at[pl.ds(wid * SEG, SEG)])


@functools.partial(
    pl.kernel,
    out_type=jax.ShapeDtypeStruct((NC, N, D), jnp.float32),
    mesh=_mesh(),
    compiler_params=pltpu.CompilerParams(needs_layout_passes=False),
    scratch_types=[
        pltpu.VMEM((EP,), jnp.int32),
        pltpu.VMEM((EP,), jnp.int32),
        pltpu.VMEM((EP,), jnp.int32),
        pltpu.VMEM((MC,), jnp.int32),
        pltpu.VMEM((MC,), jnp.int32),
        pltpu.VMEM((MC,), jnp.int32),
        pltpu.VMEM((MC,), jnp.float32),
        pltpu.VMEM((MC, D), jnp.float32),
        pltpu.VMEM_SHARED((N, D), jnp.float32),
        pltpu.SemaphoreType.DMA,
        pltpu.SemaphoreType.DMA,
    ],
)
def _main_kernel(src, dst, et, h, inv, acc_out, src_v, dst_v, et_v, idxh_v,
                 seg_v, dstc_v, invg_v, rows_v, acc_sh, sem1, sem2):
    c = lax.axis_index("c")
    s = lax.axis_index("s")
    wid = s * NC + c
    base = wid * EP

    def zrow(i, _):
        for k in range(D // L):
            rows_v[i, pl.ds(k * L, L)] = jnp.zeros((L,), jnp.float32)
        return 0

    lax.fori_loop(0, MC, zrow, 0)

    def zchunk(rep, _):
        ck = s + rep * NS

        @pl.when(ck < N // MC)
        def _():
            pltpu.sync_copy(rows_v, acc_sh.at[pl.ds(ck * MC, MC)])

        return 0

    lax.fori_loop(0, (N // MC + NS - 1) // NS, zchunk, 0)
    plsc.subcore_barrier()

    pltpu.sync_copy(src.at[pl.ds(base, EP)], src_v)
    pltpu.sync_copy(dst.at[pl.ds(base, EP)], dst_v)
    pltpu.sync_copy(et.at[pl.ds(base, EP)], et_v)

    def chunk(j, _):
        def lane(i, _):
            off = j * MC + i * L
            sv = src_v[pl.ds(off, L)]
            d = dst_v[pl.ds(off, L)]
            t = et_v[pl.ds(off, L)]
            idxh_v[pl.ds(i * L, L)] = t * N + sv
            seg_v[pl.ds(i * L, L)] = d * R + t
            dstc_v[pl.ds(i * L, L)] = d
            return 0

        lax.fori_loop(0, MC // L, lane, 0)
        g1 = pltpu.async_copy(h.at[idxh_v], rows_v, sem1)
        g2 = pltpu.async_copy(inv.at[seg_v], invg_v, sem2)
        g1.wait()
        g2.wait()

        lane_id = lax.iota(jnp.int32, L)

        def scale(g, _):
            iv16 = invg_v[pl.ds(g * L, L)]
            for l in range(L):
                e = g * L + l
                sc = jnp.sum(jnp.where(lane_id == l, iv16, 0.0))
                for k in range(D // L):
                    sl = pl.ds(k * L, L)
                    rows_v[e, sl] = rows_v[e, sl] * sc
            return 0

        lax.fori_loop(0, MC // L, scale, 0)
        pltpu.sync_copy(rows_v, acc_sh.at[dstc_v], add=True)
        return 0

    lax.fori_loop(0, EP // MC, chunk, 0)
    plsc.subcore_barrier()

    def dchunk(rep, _):
        ck = s + rep * NS

        @pl.when(ck < N // MC)
        def _():
            pltpu.sync_copy(
                acc_sh.at[pl.ds(ck * MC, MC)], acc_out.at[c, pl.ds(ck * MC, MC)]
            )

        return 0

    lax.fori_loop(0, (N // MC + NS - 1) // NS, dchunk, 0)


BN = 1000


def _hall_body(x_ref, w_ref, o_ref):
    o_ref[0] = jnp.dot(x_ref[...], w_ref[0], preferred_element_type=jnp.float32)


_hall = pl.pallas_call(
    _hall_body,
    grid=(R, N // BN),
    in_specs=[
        pl.BlockSpec((BN, D), lambda r, j: (j, 0)),
        pl.BlockSpec((1, D, D), lambda r, j: (r, 0, 0)),
    ],
    out_specs=pl.BlockSpec((1, BN, D), lambda r, j: (r, j, 0)),
    out_shape=jax.ShapeDtypeStruct((R, N, D), jnp.float32),
)

_CNT_ROWS = SEG // 128
BB = 125


def _inv_body(c_ref, o_ref):
    o_ref[...] = 1.0 / jnp.maximum(c_ref[...].sum(axis=0), 1.0)


_invk = pl.pallas_call(
    _inv_body,
    out_shape=jax.ShapeDtypeStruct((_CNT_ROWS, 128), jnp.float32),
)

BF = 2000


def _final_body(x_ref, w_ref, b_ref, a_ref, o_ref):
    o_ref[...] = (
        jnp.dot(x_ref[...], w_ref[...], preferred_element_type=jnp.float32)
        + b_ref[...]
        + a_ref[0]
        + a_ref[1]
    )


_final = pl.pallas_call(
    _final_body,
    grid=(N // BF,),
    in_specs=[
        pl.BlockSpec((BF, D), lambda j: (j, 0)),
        pl.BlockSpec((D, D), lambda j: (0, 0)),
        pl.BlockSpec((1, D), lambda j: (0, 0)),
        pl.BlockSpec((NC, BF, D), lambda j: (0, j, 0)),
    ],
    out_specs=pl.BlockSpec((BF, D), lambda j: (j, 0)),
    out_shape=jax.ShapeDtypeStruct((N, D), jnp.float32),
)


def kernel(x, edge_index, edge_type, W, W_root, bias):
    src = edge_index[0]
    dst = edge_index[1]
    cnt = _count_kernel(dst, edge_type)
    h_all = _hall(x, W).reshape(R * N, D)
    inv = _invk(cnt.reshape(NW, _CNT_ROWS, 128)).reshape(SEG)
    acc = _main_kernel(src, dst, edge_type, h_all, inv)
    return _final(x, W_root, bias.reshape(1, D), acc)


# speedup vs baseline: 24.7413x; 1.4349x over previous
import functools

import jax
import jax.numpy as jnp
import numpy as np
from jax import lax
from jax.experimental import pallas as pl
from jax.experimental.pallas import tpu as pltpu
from jax.experimental.pallas import tpu_sc as plsc

N = 10000
E = 320000
D = 128
R = 8

NC = 2
NS = 16
NW = NC * NS
L = 16

EP = E // NW
SEG = N * R
MC = 80
ARS = N // NS


_mesh = functools.partial(
    plsc.VectorSubcoreMesh,
    core_axis_name="c",
    subcore_axis_name="s",
    num_cores=NC,
    num_subcores=NS,
)


@functools.partial(
    pl.kernel,
    out_type=jax.ShapeDtypeStruct((NW * SEG,), jnp.float32),
    mesh=_mesh(),
    compiler_params=pltpu.CompilerParams(needs_layout_passes=False),
    scratch_types=[
        pltpu.VMEM((EP,), jnp.int32),
        pltpu.VMEM((EP,), jnp.int32),
        pltpu.VMEM((SEG,), jnp.float32),
    ],
)
def _count_kernel(dst, et, cnt_out, dst_v, et_v, cnt_v):
    c = lax.axis_index("c")
    s = lax.axis_index("s")
    wid = s * NC + c
    base = wid * EP

    def zero(i, _):
        cnt_v[pl.ds(i * L, L)] = jnp.zeros((L,), jnp.float32)
        return 0

    lax.fori_loop(0, SEG // L, zero, 0)
    pltpu.sync_copy(dst.at[pl.ds(base, EP)], dst_v)
    pltpu.sync_copy(et.at[pl.ds(base, EP)], et_v)

    ones16 = jnp.ones((L,), jnp.float32)

    def lane(i, _):
        off = i * L
        d = dst_v[pl.ds(off, L)]
        t = et_v[pl.ds(off, L)]
        plsc.addupdate_scatter(cnt_v, [d * R + t], ones16)
        return 0

    lax.fori_loop(0, EP // L, lane, 0)
    pltpu.sync_copy(cnt_v, cnt_out.at[pl.ds(wid * SEG, SEG)])


NCH = EP // MC


@functools.partial(
    pl.kernel,
    out_type=jax.ShapeDtypeStruct((NC, N, D), jnp.float32),
    mesh=_mesh(),
    compiler_params=pltpu.CompilerParams(needs_layout_passes=False),
    scratch_types=[
        [pltpu.VMEM((MC,), jnp.int32)] * 2,
        [pltpu.VMEM((MC,), jnp.int32)] * 2,
        [pltpu.VMEM((MC,), jnp.int32)] * 2,
        [pltpu.VMEM((MC,), jnp.int32)] * 2,
        [pltpu.VMEM((MC,), jnp.int32)] * 2,
        [pltpu.VMEM((MC,), jnp.int32)] * 2,
        [pltpu.VMEM((MC,), jnp.float32)] * 2,
        [pltpu.VMEM((MC, D), jnp.float32)] * 2,
        pltpu.VMEM_SHARED((N, D), jnp.float32),
        [pltpu.SemaphoreType.DMA] * 2,
        [pltpu.SemaphoreType.DMA] * 2,
    ],
)
def _main_kernel(src, dst, et, h, inv, acc_out, srcb, dstb, etb, idxhb,
                 segb, dstcb, invgb, rowsb, acc_sh, esem, gsem):
    c = lax.axis_index("c")
    s = lax.axis_index("s")
    wid = s * NC + c
    base = wid * EP

    def zrow(i, _):
        for k in range(D // L):
            rowsb[0][i, pl.ds(k * L, L)] = jnp.zeros((L,), jnp.float32)
        return 0

    lax.fori_loop(0, MC, zrow, 0)

    def zchunk(rep, _):
        ck = s + rep * NS

        @pl.when(ck < N // MC)
        def _():
            pltpu.sync_copy(rowsb[0], acc_sh.at[pl.ds(ck * MC, MC)])

        return 0

    lax.fori_loop(0, (N // MC + NS - 1) // NS, zchunk, 0)
    plsc.subcore_barrier()

    def fire_edges(k, b):
        @pl.when(k < NCH)
        def _():
            bk = base + k * MC
            pltpu.async_copy(src.at[pl.ds(bk, MC)], srcb[b], esem[b])
            pltpu.async_copy(dst.at[pl.ds(bk, MC)], dstb[b], esem[b])
            pltpu.async_copy(et.at[pl.ds(bk, MC)], etb[b], esem[b])

    def wait_edges(b):
        pltpu.make_async_copy(src.at[pl.ds(base, MC)], srcb[b], esem[b]).wait()
        pltpu.make_async_copy(dst.at[pl.ds(base, MC)], dstb[b], esem[b]).wait()
        pltpu.make_async_copy(et.at[pl.ds(base, MC)], etb[b], esem[b]).wait()

    def prep_and_fire_gathers(b):
        def lane(i, _):
            off = i * L
            sv = srcb[b][pl.ds(off, L)]
            d = dstb[b][pl.ds(off, L)]
            t = etb[b][pl.ds(off, L)]
            idxhb[b][pl.ds(off, L)] = t * N + sv
            segb[b][pl.ds(off, L)] = d * R + t
            dstcb[b][pl.ds(off, L)] = d
            return 0

        lax.fori_loop(0, MC // L, lane, 0)
        pltpu.async_copy(h.at[idxhb[b]], rowsb[b], gsem[b])
        pltpu.async_copy(inv.at[segb[b]], invgb[b], gsem[b])

    def wait_gathers(b):
        pltpu.make_async_copy(h.at[idxhb[b]], rowsb[b], gsem[b]).wait()
        pltpu.make_async_copy(inv.at[segb[b]], invgb[b], gsem[b]).wait()

    lane_id = lax.iota(jnp.int32, L)

    def consume(b):
        wait_gathers(b)

        def scale(g, _):
            iv16 = invgb[b][pl.ds(g * L, L)]
            for l in range(L):
                e = g * L + l
                sc = jnp.sum(jnp.where(lane_id == l, iv16, 0.0))
                for k in range(D // L):
                    sl = pl.ds(k * L, L)
                    rowsb[b][e, sl] = rowsb[b][e, sl] * sc
            return 0

        lax.fori_loop(0, MC // L, scale, 0)
        pltpu.sync_copy(rowsb[b], acc_sh.at[dstcb[b]], add=True)

    fire_edges(0, 0)
    fire_edges(1, 1)
    wait_edges(0)
    prep_and_fire_gathers(0)

    def pair(j2, _):
        for b in range(2):
            k = j2 * 2 + b
            fire_edges(k + 2, b)
            ob = 1 - b

            @pl.when(k + 1 < NCH)
            def _():
                wait_edges(ob)
                prep_and_fire_gathers(ob)

            @pl.when(k < NCH)
            def _():
                consume(b)
        return 0

    lax.fori_loop(0, (NCH + 1) // 2, pair, 0)
    plsc.subcore_barrier()

    def dchunk(rep, _):
        ck = s + rep * NS

        @pl.when(ck < N // MC)
        def _():
            pltpu.sync_copy(
                acc_sh.at[pl.ds(ck * MC, MC)], acc_out.at[c, pl.ds(ck * MC, MC)]
            )

        return 0

    lax.fori_loop(0, (N // MC + NS - 1) // NS, dchunk, 0)


BN = 1000


def _hall_body(x_ref, w_ref, o_ref):
    o_ref[0] = jnp.dot(x_ref[...], w_ref[0], preferred_element_type=jnp.float32)


_hall = pl.pallas_call(
    _hall_body,
    grid=(R, N // BN),
    in_specs=[
        pl.BlockSpec((BN, D), lambda r, j: (j, 0)),
        pl.BlockSpec((1, D, D), lambda r, j: (r, 0, 0)),
    ],
    out_specs=pl.BlockSpec((1, BN, D), lambda r, j: (r, j, 0)),
    out_shape=jax.ShapeDtypeStruct((R, N, D), jnp.float32),
)

_CNT_ROWS = SEG // 128
BB = 125


def _inv_body(c_ref, o_ref):
    o_ref[...] = 1.0 / jnp.maximum(c_ref[...].sum(axis=0), 1.0)


_invk = pl.pallas_call(
    _inv_body,
    out_shape=jax.ShapeDtypeStruct((_CNT_ROWS, 128), jnp.float32),
)

BF = 2000


def _final_body(x_ref, w_ref, b_ref, a_ref, o_ref):
    o_ref[...] = (
        jnp.dot(x_ref[...], w_ref[...], preferred_element_type=jnp.float32)
        + b_ref[...]
        + a_ref[0]
        + a_ref[1]
    )


_final = pl.pallas_call(
    _final_body,
    grid=(N // BF,),
    in_specs=[
        pl.BlockSpec((BF, D), lambda j: (j, 0)),
        pl.BlockSpec((D, D), lambda j: (0, 0)),
        pl.BlockSpec((1, D), lambda j: (0, 0)),
        pl.BlockSpec((NC, BF, D), lambda j: (0, j, 0)),
    ],
    out_specs=pl.BlockSpec((BF, D), lambda j: (j, 0)),
    out_shape=jax.ShapeDtypeStruct((N, D), jnp.float32),
)


def kernel(x, edge_index, edge_type, W, W_root, bias):
    src = edge_index[0]
    dst = edge_index[1]
    cnt = _count_kernel(dst, edge_type)
    h_all = _hall(x, W).reshape(R * N, D)
    inv = _invk(cnt.reshape(NW, _CNT_ROWS, 128)).reshape(SEG)
    acc = _main_kernel(src, dst, edge_type, h_all, inv)
    return _final(x, W_root, bias.reshape(1, D), acc)
